# baseline (speedup 1.0000x reference)
import jax
import jax.numpy as jnp
from jax import lax
from jax.experimental import pallas as pl

B, N, DN, DC, H = 16384, 50, 16, 128, 64
BLK = 256


def _tc_kernel(idx_ref, adj_ref, node_ref, cent_ref,
               Wg1_ref, bg1_ref, Wg2_ref, bg2_ref,
               Wm1a_ref, Wm1b_ref, bm1_ref, Wm2_ref, bm2_ref,
               WvT_ref, bv_ref, out_ref):
    adj = adj_ref[...].reshape(BLK, N, N)
    node = node_ref[...].reshape(BLK, N, DN)

    p3 = lax.dot_general(node, Wg1_ref[...], (((2,), (0,)), ((), ())),
                         preferred_element_type=jnp.float32)

    m1 = lax.dot_general(adj, p3, (((2,), (1,)), ((0,), (0,))),
                         preferred_element_type=jnp.float32)
    h1 = jax.nn.relu(m1 + bg1_ref[...])

    idx = idx_ref[...]
    iota = lax.broadcasted_iota(jnp.int32, (BLK, N), 1)
    onehot = (iota == idx).astype(jnp.float32).reshape(BLK, 1, N)
    a_ego = lax.dot_general(onehot, adj, (((2,), (1,)), ((0,), (0,))),
                            preferred_element_type=jnp.float32)

    m2e = lax.dot_general(a_ego, h1, (((2,), (1,)), ((0,), (0,))),
                          preferred_element_type=jnp.float32).reshape(BLK, H)
    h2e = jax.nn.relu(jnp.dot(m2e, Wg2_ref[...],
                              preferred_element_type=jnp.float32) + bg2_ref[...])

    x = jax.nn.relu(jnp.dot(cent_ref[...], Wm1a_ref[...], preferred_element_type=jnp.float32)
                    + jnp.dot(h2e, Wm1b_ref[...], preferred_element_type=jnp.float32)
                    + bm1_ref[...])
    x = jax.nn.relu(jnp.dot(x, Wm2_ref[...], preferred_element_type=jnp.float32)
                    + bm2_ref[...])
    out_ref[...] = jnp.sum(x * WvT_ref[...], axis=1, keepdims=True) + bv_ref[...]


def kernel(cent_obs, node_obs, adj, agent_id,
           W_gnn1, b_gnn1, W_gnn2, b_gnn2,
           W_mlp1, b_mlp1, W_mlp2, b_mlp2,
           W_v, b_v):
    idx = agent_id.astype(jnp.int32).reshape(B, 1)
    grid = (B // BLK,)

    const = lambda *shape: pl.BlockSpec(shape, lambda i: (0,) * len(shape))
    out = pl.pallas_call(
        _tc_kernel,
        grid=grid,
        in_specs=[
            pl.BlockSpec((BLK, 1), lambda i: (i, 0)),
            pl.BlockSpec((BLK, N * N), lambda i: (i, 0)),
            pl.BlockSpec((BLK, N * DN), lambda i: (i, 0)),
            pl.BlockSpec((BLK, DC), lambda i: (i, 0)),
            const(DN, H), const(1, H),
            const(H, H), const(1, H),
            const(DC, H), const(H, H), const(1, H),
            const(H, H), const(1, H),
            const(1, H), const(1, 1),
        ],
        out_specs=pl.BlockSpec((BLK, 1), lambda i: (i, 0)),
        out_shape=jax.ShapeDtypeStruct((B, 1), jnp.float32),
    )(idx, adj.reshape(B, N * N), node_obs.reshape(B, N * DN), cent_obs,
      W_gnn1, b_gnn1.reshape(1, H), W_gnn2, b_gnn2.reshape(1, H),
      W_mlp1[:DC], W_mlp1[DC:], b_mlp1.reshape(1, H),
      W_mlp2, b_mlp2.reshape(1, H),
      W_v.reshape(1, H), b_v.reshape(1, 1))
    return out


# speedup vs baseline: 1.0503x; 1.0503x over previous
import jax
import jax.numpy as jnp
from jax import lax
from jax.experimental import pallas as pl

B, N, DN, DC, H = 16384, 50, 16, 128, 64
BLK = 256


def _tc_kernel(idx_ref, adj_ref, node_ref, cent_ref, Wp_ref, out_ref):
    W = Wp_ref[...]
    Wg1 = W[0:16]; Wg2 = W[16:80]; Wm1a = W[80:208]; Wm1b = W[208:272]
    Wm2 = W[272:336]
    bg1 = W[336:337]; bg2 = W[337:338]; bm1 = W[338:339]; bm2 = W[339:340]
    WvT = W[340:341]; bv = W[341:342, 0:1]
    adj = adj_ref[...]
    node = node_ref[...].reshape(BLK, N, DN)

    p3 = lax.dot_general(node, Wg1, (((2,), (0,)), ((), ())),
                         preferred_element_type=jnp.float32)

    m1 = lax.dot_general(adj, p3, (((2,), (1,)), ((0,), (0,))),
                         preferred_element_type=jnp.float32)
    h1 = jax.nn.relu(m1 + bg1)

    idx = idx_ref[...]
    iota = lax.broadcasted_iota(jnp.int32, (BLK, N), 1)
    onehot = (iota == idx).astype(jnp.float32).reshape(BLK, 1, N)
    a_ego = lax.dot_general(onehot, adj, (((2,), (1,)), ((0,), (0,))),
                            preferred_element_type=jnp.float32)

    m2e = lax.dot_general(a_ego, h1, (((2,), (1,)), ((0,), (0,))),
                          preferred_element_type=jnp.float32).reshape(BLK, H)
    h2e = jax.nn.relu(jnp.dot(m2e, Wg2,
                              preferred_element_type=jnp.float32) + bg2)

    x = jax.nn.relu(jnp.dot(cent_ref[...], Wm1a, preferred_element_type=jnp.float32)
                    + jnp.dot(h2e, Wm1b, preferred_element_type=jnp.float32)
                    + bm1)
    x = jax.nn.relu(jnp.dot(x, Wm2, preferred_element_type=jnp.float32)
                    + bm2)
    out_ref[...] = jnp.sum(x * WvT, axis=1, keepdims=True) + bv


def kernel(cent_obs, node_obs, adj, agent_id,
           W_gnn1, b_gnn1, W_gnn2, b_gnn2,
           W_mlp1, b_mlp1, W_mlp2, b_mlp2,
           W_v, b_v):
    idx = agent_id.astype(jnp.int32).reshape(B, 1)
    Wp = jnp.concatenate([
        W_gnn1, W_gnn2, W_mlp1[:DC], W_mlp1[DC:], W_mlp2,
        b_gnn1.reshape(1, H), b_gnn2.reshape(1, H), b_mlp1.reshape(1, H),
        b_mlp2.reshape(1, H), W_v.reshape(1, H),
        jnp.pad(b_v.reshape(1, 1), ((0, 0), (0, H - 1))),
        jnp.zeros((2, H), jnp.float32)], axis=0)
    grid = (B // BLK,)

    const = lambda *shape: pl.BlockSpec(shape, lambda i: (0,) * len(shape))
    out = pl.pallas_call(
        _tc_kernel,
        grid=grid,
        in_specs=[
            pl.BlockSpec((BLK, 1), lambda i: (i, 0)),
            pl.BlockSpec((BLK, N, N), lambda i: (i, 0, 0)),
            pl.BlockSpec((BLK, N * DN), lambda i: (i, 0)),
            pl.BlockSpec((BLK, DC), lambda i: (i, 0)),
            const(344, H),
        ],
        out_specs=pl.BlockSpec((BLK, 1), lambda i: (i, 0)),
        out_shape=jax.ShapeDtypeStruct((B, 1), jnp.float32),
    )(idx, adj, node_obs.reshape(B, N * DN), cent_obs, Wp)
    return out


# speedup vs baseline: 1.0707x; 1.0194x over previous
import jax
import jax.numpy as jnp
from jax import lax
from jax.experimental import pallas as pl

B, N, DN, DC, H = 16384, 50, 16, 128, 64
BLK = 512


def _tc_kernel(idx_ref, adj_ref, node_ref, cent_ref,
               Wg1_ref, bg1_ref, Wg2_ref, bg2_ref,
               Wm1a_ref, Wm1b_ref, bm1_ref, Wm2_ref, bm2_ref,
               WvT_ref, bv_ref, out_ref):
    adj = adj_ref[...]
    node = node_ref[...].reshape(BLK, N, DN)

    p3 = lax.dot_general(node, Wg1_ref[...], (((2,), (0,)), ((), ())),
                         preferred_element_type=jnp.float32)

    m1 = lax.dot_general(adj, p3, (((2,), (1,)), ((0,), (0,))),
                         preferred_element_type=jnp.float32)
    h1 = jax.nn.relu(m1 + bg1_ref[...])

    idx = idx_ref[...]
    iota = lax.broadcasted_iota(jnp.int32, (BLK, N), 1)
    onehot = (iota == idx).astype(jnp.float32).reshape(BLK, 1, N)
    a_ego = lax.dot_general(onehot, adj, (((2,), (1,)), ((0,), (0,))),
                            preferred_element_type=jnp.float32)

    m2e = lax.dot_general(a_ego, h1, (((2,), (1,)), ((0,), (0,))),
                          preferred_element_type=jnp.float32).reshape(BLK, H)
    h2e = jax.nn.relu(jnp.dot(m2e, Wg2_ref[...],
                              preferred_element_type=jnp.float32) + bg2_ref[...])

    x = jax.nn.relu(jnp.dot(cent_ref[...], Wm1a_ref[...], preferred_element_type=jnp.float32)
                    + jnp.dot(h2e, Wm1b_ref[...], preferred_element_type=jnp.float32)
                    + bm1_ref[...])
    x = jax.nn.relu(jnp.dot(x, Wm2_ref[...], preferred_element_type=jnp.float32)
                    + bm2_ref[...])
    out_ref[...] = jnp.sum(x * WvT_ref[...], axis=1, keepdims=True) + bv_ref[...]


def kernel(cent_obs, node_obs, adj, agent_id,
           W_gnn1, b_gnn1, W_gnn2, b_gnn2,
           W_mlp1, b_mlp1, W_mlp2, b_mlp2,
           W_v, b_v):
    idx = agent_id.astype(jnp.int32).reshape(B, 1)
    grid = (B // BLK,)

    const = lambda *shape: pl.BlockSpec(shape, lambda i: (0,) * len(shape))
    out = pl.pallas_call(
        _tc_kernel,
        grid=grid,
        in_specs=[
            pl.BlockSpec((BLK, 1), lambda i: (i, 0)),
            pl.BlockSpec((BLK, N, N), lambda i: (i, 0, 0)),
            pl.BlockSpec((BLK, N * DN), lambda i: (i, 0)),
            pl.BlockSpec((BLK, DC), lambda i: (i, 0)),
            const(DN, H), const(1, H),
            const(H, H), const(1, H),
            const(DC, H), const(H, H), const(1, H),
            const(H, H), const(1, H),
            const(1, H), const(1, 1),
        ],
        out_specs=pl.BlockSpec((BLK, 1), lambda i: (i, 0)),
        out_shape=jax.ShapeDtypeStruct((B, 1), jnp.float32),
    )(idx, adj, node_obs.reshape(B, N * DN), cent_obs,
      W_gnn1, b_gnn1.reshape(1, H), W_gnn2, b_gnn2.reshape(1, H),
      W_mlp1[:DC], W_mlp1[DC:], b_mlp1.reshape(1, H),
      W_mlp2, b_mlp2.reshape(1, H),
      W_v.reshape(1, H), b_v.reshape(1, 1))
    return out
